# baseline (speedup 1.0000x reference)
import functools

import jax
import jax.numpy as jnp
from jax import lax
from jax.experimental import pallas as pl
from jax.experimental.pallas import tpu as pltpu
from jax.experimental.pallas import tpu_sc as plsc

_NUM_CORES = 2
_NUM_SUBCORES = 16
_NUM_WORKERS = _NUM_CORES * _NUM_SUBCORES
_CHUNK = 512
_PREP_BLOCK = 2048
_SEL_BLOCK = 128
_FPAD = 32


def _tc_prep(xp):
    batch = xp.shape[0]
    n_blocks = batch // _PREP_BLOCK
    rows_block = _PREP_BLOCK * _FPAD // 128

    def prep_kernel(x_ref, out_ref):
        a = x_ref[...] >> 2
        out_ref[...] = jnp.concatenate(
            [a[0::4, :], a[1::4, :], a[2::4, :], a[3::4, :]], axis=1
        )

    return pl.pallas_call(
        prep_kernel,
        grid=(n_blocks,),
        in_specs=[pl.BlockSpec((_PREP_BLOCK, _FPAD), lambda i: (i, 0))],
        out_specs=pl.BlockSpec((rows_block, 128), lambda i: (i, 0)),
        out_shape=jax.ShapeDtypeStruct((batch * _FPAD // 128, 128), jnp.int32),
    )(xp)


def _sc_gather(table4, idx4):
    num_rows = idx4.shape[0]
    rows_per_worker = num_rows // _NUM_WORKERS
    n_chunks = rows_per_worker // _CHUNK
    mesh = plsc.VectorSubcoreMesh(core_axis_name="c", subcore_axis_name="s")

    @functools.partial(
        pl.kernel,
        mesh=mesh,
        out_type=jax.ShapeDtypeStruct((num_rows, 128), jnp.float32),
        scratch_types=[
            pltpu.VMEM((_CHUNK,), jnp.int32),
            pltpu.VMEM((_CHUNK, 128), jnp.float32),
            pltpu.SemaphoreType.DMA,
        ],
    )
    def gather_kernel(table_hbm, idx_hbm, out_hbm, idx_v, lines_v, sem):
        wid = lax.axis_index("s") * _NUM_CORES + lax.axis_index("c")
        base = wid * rows_per_worker

        @pl.loop(0, n_chunks)
        def _(c):
            off = base + c * _CHUNK
            pltpu.sync_copy(idx_hbm.at[pl.ds(off, _CHUNK)], idx_v)
            pltpu.async_copy(table_hbm.at[idx_v], lines_v, sem).wait()
            pltpu.sync_copy(lines_v, out_hbm.at[pl.ds(off, _CHUNK)])

    return gather_kernel(table4, idx4)


def _tc_select(lines, rem, fields):
    batch = rem.shape[0]
    dim = 32
    n_blocks = batch // _SEL_BLOCK

    def select_kernel(lines_ref, rem_ref, out_ref):
        a = lines_ref[...].reshape(_SEL_BLOCK, _FPAD, 128)
        r = rem_ref[...].reshape(_SEL_BLOCK, _FPAD, 1)
        w = jnp.where(
            r < 2,
            jnp.where(r == 0, a[:, :, 0:dim], a[:, :, dim : 2 * dim]),
            jnp.where(r == 2, a[:, :, 2 * dim : 3 * dim], a[:, :, 3 * dim :]),
        )
        out_ref[...] = w[:, :fields, :]

    return pl.pallas_call(
        select_kernel,
        grid=(n_blocks,),
        in_specs=[
            pl.BlockSpec((_SEL_BLOCK * _FPAD, 128), lambda i: (i, 0)),
            pl.BlockSpec((_SEL_BLOCK, _FPAD), lambda i: (i, 0)),
        ],
        out_specs=pl.BlockSpec((_SEL_BLOCK, fields, dim), lambda i: (i, 0, 0)),
        out_shape=jax.ShapeDtypeStruct((batch, fields, dim), jnp.float32),
    )(lines, rem)


def kernel(x, table):
    batch, fields = x.shape
    xp = jnp.pad(x.astype(jnp.int32), ((0, 0), (0, _FPAD - fields)))
    idx4 = (xp >> 2).reshape(batch * _FPAD)
    rem = xp & 3
    table4 = table.reshape(table.shape[0] // 4, 128)
    lines = _sc_gather(table4, idx4)
    return _tc_select(lines, rem, fields)


# speedup vs baseline: 4.4835x; 4.4835x over previous
import functools

import jax
import jax.numpy as jnp
from jax import lax
from jax.experimental import pallas as pl
from jax.experimental.pallas import tpu as pltpu
from jax.experimental.pallas import tpu_sc as plsc

_NUM_CORES = 2
_NUM_SUBCORES = 16
_NUM_WORKERS = _NUM_CORES * _NUM_SUBCORES
_CHUNK = 512
_GROUP = 128
_FPAD = 32


def _tc_prep(x):
    batch, fields = x.shape
    n_blocks = batch // _GROUP

    def prep_kernel(x_ref, idx_ref, rem_ref):
        a = x_ref[...]
        ap = jnp.concatenate([a, a[:, fields - (_FPAD - fields) :]], axis=1)
        rem_ref[...] = ap & 3
        line = ap >> 2
        idx_ref[...] = jnp.concatenate(
            [line[q : q + _FPAD, :] for q in range(0, _GROUP, _FPAD)], axis=1
        )

    return pl.pallas_call(
        prep_kernel,
        grid=(n_blocks,),
        in_specs=[pl.BlockSpec((_GROUP, fields), lambda i: (i, 0))],
        out_specs=[
            pl.BlockSpec((_FPAD, 128), lambda i: (i, 0)),
            pl.BlockSpec((_GROUP, _FPAD), lambda i: (i, 0)),
        ],
        out_shape=[
            jax.ShapeDtypeStruct((batch * _FPAD // 128, 128), jnp.int32),
            jax.ShapeDtypeStruct((batch, _FPAD), jnp.int32),
        ],
    )(x)


def _sc_gather(table4, idx4):
    num_rows = idx4.shape[0]
    rows_per_worker = num_rows // _NUM_WORKERS
    n_chunks = rows_per_worker // _CHUNK
    mesh = plsc.VectorSubcoreMesh(core_axis_name="c", subcore_axis_name="s")

    @functools.partial(
        pl.kernel,
        mesh=mesh,
        out_type=jax.ShapeDtypeStruct((num_rows, 128), jnp.float32),
        scratch_types=[
            pltpu.VMEM((_CHUNK,), jnp.int32),
            pltpu.VMEM((_CHUNK, 128), jnp.float32),
            pltpu.SemaphoreType.DMA,
        ],
    )
    def gather_kernel(table_hbm, idx_hbm, out_hbm, idx_v, lines_v, sem):
        wid = lax.axis_index("s") * _NUM_CORES + lax.axis_index("c")
        base = wid * rows_per_worker

        @pl.loop(0, n_chunks)
        def _(c):
            off = base + c * _CHUNK
            pltpu.sync_copy(idx_hbm.at[pl.ds(off, _CHUNK)], idx_v)
            pltpu.async_copy(table_hbm.at[idx_v], lines_v, sem).wait()
            pltpu.sync_copy(lines_v, out_hbm.at[pl.ds(off, _CHUNK)])

    return gather_kernel(table4, idx4)


def _tc_select(lines, rem, fields):
    batch = rem.shape[0]
    dim = 32
    n_blocks = batch // _GROUP
    lines_rows = _GROUP * _FPAD

    def select_kernel(lines_ref, rem_ref, out_ref):
        a = lines_ref[...].reshape(_FPAD, 4, _FPAD, 128)
        a = a.transpose(1, 0, 2, 3).reshape(_GROUP, _FPAD, 128)
        r = rem_ref[...].reshape(_GROUP, _FPAD, 1)
        w = jnp.where(
            r < 2,
            jnp.where(r == 0, a[:, :, 0:dim], a[:, :, dim : 2 * dim]),
            jnp.where(r == 2, a[:, :, 2 * dim : 3 * dim], a[:, :, 3 * dim :]),
        )
        out_ref[...] = w[:, :fields, :]

    return pl.pallas_call(
        select_kernel,
        grid=(n_blocks,),
        in_specs=[
            pl.BlockSpec((lines_rows, 128), lambda i: (i, 0)),
            pl.BlockSpec((_GROUP, _FPAD), lambda i: (i, 0)),
        ],
        out_specs=pl.BlockSpec((_GROUP, fields, dim), lambda i: (i, 0, 0)),
        out_shape=jax.ShapeDtypeStruct((batch, fields, dim), jnp.float32),
    )(lines, rem)


def kernel(x, table):
    batch, fields = x.shape
    idx4_2d, rem = _tc_prep(x.astype(jnp.int32))
    idx4 = idx4_2d.reshape(batch * _FPAD)
    table4 = table.reshape(table.shape[0] // 4, 128)
    lines = _sc_gather(table4, idx4)
    return _tc_select(lines, rem, fields)


# speedup vs baseline: 4.4960x; 1.0028x over previous
import functools

import jax
import jax.numpy as jnp
from jax import lax
from jax.experimental import pallas as pl
from jax.experimental.pallas import tpu as pltpu
from jax.experimental.pallas import tpu_sc as plsc

_NUM_CORES = 2
_NUM_SUBCORES = 16
_NUM_WORKERS = _NUM_CORES * _NUM_SUBCORES
_CHUNK = 512
_GROUP = 128
_FPAD = 32


def _tc_prep(x):
    batch, fields = x.shape
    n_blocks = batch // _GROUP

    def prep_kernel(x_ref, idx_ref, rem_ref):
        a = x_ref[...]
        ap = jnp.concatenate([a, a[:, fields - (_FPAD - fields) :]], axis=1)
        rem_ref[...] = ap & 3
        line = ap >> 2
        idx_ref[...] = jnp.concatenate(
            [line[q : q + _FPAD, :] for q in range(0, _GROUP, _FPAD)], axis=1
        ).reshape(_FPAD, 1, 128)

    return pl.pallas_call(
        prep_kernel,
        grid=(n_blocks,),
        in_specs=[pl.BlockSpec((_GROUP, fields), lambda i: (i, 0))],
        out_specs=[
            pl.BlockSpec((_FPAD, 1, 128), lambda i: (i, 0, 0)),
            pl.BlockSpec((_GROUP, _FPAD), lambda i: (i, 0)),
        ],
        out_shape=[
            jax.ShapeDtypeStruct((batch * _FPAD // 128, 1, 128), jnp.int32),
            jax.ShapeDtypeStruct((batch, _FPAD), jnp.int32),
        ],
    )(x)


def _sc_gather(table4, idx4_2d):
    idx_rows = idx4_2d.shape[0]
    num_rows = idx_rows * 128
    chunk_idx_rows = _CHUNK // 128
    rows_per_worker = num_rows // _NUM_WORKERS
    n_chunks = rows_per_worker // _CHUNK
    mesh = plsc.VectorSubcoreMesh(core_axis_name="c", subcore_axis_name="s")

    @functools.partial(
        pl.kernel,
        mesh=mesh,
        out_type=jax.ShapeDtypeStruct((num_rows, 128), jnp.float32),
        scratch_types=[
            pltpu.VMEM((8, 1, 128), jnp.int32),
            pltpu.VMEM((_CHUNK, 128), jnp.float32),
            pltpu.SemaphoreType.DMA,
        ],
    )
    def gather_kernel(table_hbm, idx_hbm, out_hbm, idx_v, lines_v, sem):
        wid = lax.axis_index("s") * _NUM_CORES + lax.axis_index("c")
        base = wid * rows_per_worker

        @pl.loop(0, n_chunks // 2)
        def _(c):
            off = base + c * 2 * _CHUNK
            pltpu.sync_copy(idx_hbm.at[pl.ds(off // 128, 8), :, :], idx_v)
            for half in range(2):
                waits = [
                    pltpu.async_copy(
                        table_hbm.at[idx_v.at[4 * half + j, 0]],
                        lines_v.at[pl.ds(j * 128, 128)],
                        sem,
                    )
                    for j in range(chunk_idx_rows)
                ]
                for w in waits:
                    w.wait()
                pltpu.sync_copy(
                    lines_v, out_hbm.at[pl.ds(off + half * _CHUNK, _CHUNK)]
                )

    return gather_kernel(table4, idx4_2d)


def _tc_select(lines, rem, fields):
    batch = rem.shape[0]
    dim = 32
    n_blocks = batch // _GROUP
    lines_rows = _GROUP * _FPAD

    def select_kernel(lines_ref, rem_ref, out_ref):
        a = lines_ref[...].reshape(_FPAD, 4, _FPAD, 128)
        a = a.transpose(1, 0, 2, 3).reshape(_GROUP, _FPAD, 128)
        r = rem_ref[...].reshape(_GROUP, _FPAD, 1)
        w = jnp.where(
            r < 2,
            jnp.where(r == 0, a[:, :, 0:dim], a[:, :, dim : 2 * dim]),
            jnp.where(r == 2, a[:, :, 2 * dim : 3 * dim], a[:, :, 3 * dim :]),
        )
        out_ref[...] = w[:, :fields, :]

    return pl.pallas_call(
        select_kernel,
        grid=(n_blocks,),
        in_specs=[
            pl.BlockSpec((lines_rows, 128), lambda i: (i, 0)),
            pl.BlockSpec((_GROUP, _FPAD), lambda i: (i, 0)),
        ],
        out_specs=pl.BlockSpec((_GROUP, fields, dim), lambda i: (i, 0, 0)),
        out_shape=jax.ShapeDtypeStruct((batch, fields, dim), jnp.float32),
    )(lines, rem)


def kernel(x, table):
    batch, fields = x.shape
    idx4_2d, rem = _tc_prep(x.astype(jnp.int32))
    table4 = table.reshape(table.shape[0] // 4, 128)
    lines = _sc_gather(table4, idx4_2d)
    return _tc_select(lines, rem, fields)


# speedup vs baseline: 5.0226x; 1.1171x over previous
import functools

import jax
import jax.numpy as jnp
from jax import lax
from jax.experimental import pallas as pl
from jax.experimental.pallas import tpu as pltpu
from jax.experimental.pallas import tpu_sc as plsc

_NUM_CORES = 2
_NUM_SUBCORES = 16
_NUM_WORKERS = _NUM_CORES * _NUM_SUBCORES
_CHUNK = 512
_GROUP = 128
_FPAD = 32


def _tc_prep(x):
    batch, fields = x.shape
    n_blocks = batch // _GROUP

    def prep_kernel(x_ref, idx_ref, rem_ref):
        a = x_ref[...]
        ap = jnp.concatenate([a, a[:, fields - (_FPAD - fields) :]], axis=1)
        rem_ref[...] = (ap >> 10) & 3
        line = ((ap >> 12) << 10) | (ap & 1023)
        idx_ref[...] = jnp.concatenate(
            [line[q : q + _FPAD, :] for q in range(0, _GROUP, _FPAD)], axis=1
        ).reshape(_FPAD, 1, 128)

    return pl.pallas_call(
        prep_kernel,
        grid=(n_blocks,),
        in_specs=[pl.BlockSpec((_GROUP, fields), lambda i: (i, 0))],
        out_specs=[
            pl.BlockSpec((_FPAD, 1, 128), lambda i: (i, 0, 0)),
            pl.BlockSpec((_GROUP, _FPAD), lambda i: (i, 0)),
        ],
        out_shape=[
            jax.ShapeDtypeStruct((batch * _FPAD // 128, 1, 128), jnp.int32),
            jax.ShapeDtypeStruct((batch, _FPAD), jnp.int32),
        ],
    )(x)


_PACK_COLS = 4096


def _tc_pack(tt):
    dim, num_emb = tt.shape
    n_blocks = pl.cdiv(num_emb, _PACK_COLS)
    lines_block = _PACK_COLS // 4

    def pack_kernel(t_ref, out_ref):
        a = t_ref[...].T
        out_ref[...] = jnp.concatenate(
            [a[q : q + lines_block, :] for q in range(0, _PACK_COLS, lines_block)],
            axis=1,
        )

    return pl.pallas_call(
        pack_kernel,
        grid=(n_blocks,),
        in_specs=[pl.BlockSpec((dim, _PACK_COLS), lambda i: (0, i))],
        out_specs=pl.BlockSpec((lines_block, 128), lambda i: (i, 0)),
        out_shape=jax.ShapeDtypeStruct((n_blocks * lines_block, 128), jnp.float32),
    )(tt)


def _sc_gather(table4, idx4_2d):
    idx_rows = idx4_2d.shape[0]
    num_rows = idx_rows * 128
    chunk_idx_rows = _CHUNK // 128
    rows_per_worker = num_rows // _NUM_WORKERS
    n_chunks = rows_per_worker // _CHUNK
    mesh = plsc.VectorSubcoreMesh(core_axis_name="c", subcore_axis_name="s")

    @functools.partial(
        pl.kernel,
        mesh=mesh,
        out_type=jax.ShapeDtypeStruct((num_rows, 128), jnp.float32),
        scratch_types=[
            pltpu.VMEM((8, 1, 128), jnp.int32),
            pltpu.VMEM((_CHUNK, 128), jnp.float32),
            pltpu.SemaphoreType.DMA,
        ],
    )
    def gather_kernel(table_hbm, idx_hbm, out_hbm, idx_v, lines_v, sem):
        wid = lax.axis_index("s") * _NUM_CORES + lax.axis_index("c")
        base = wid * rows_per_worker

        @pl.loop(0, n_chunks // 2)
        def _(c):
            off = base + c * 2 * _CHUNK
            pltpu.sync_copy(idx_hbm.at[pl.ds(off // 128, 8), :, :], idx_v)
            for half in range(2):
                waits = [
                    pltpu.async_copy(
                        table_hbm.at[idx_v.at[4 * half + j, 0]],
                        lines_v.at[pl.ds(j * 128, 128)],
                        sem,
                    )
                    for j in range(chunk_idx_rows)
                ]
                for w in waits:
                    w.wait()
                pltpu.sync_copy(
                    lines_v, out_hbm.at[pl.ds(off + half * _CHUNK, _CHUNK)]
                )

    return gather_kernel(table4, idx4_2d)


def _tc_select(lines, rem, fields):
    batch = rem.shape[0]
    dim = 32
    n_blocks = batch // _GROUP
    lines_rows = _GROUP * _FPAD

    def select_kernel(lines_ref, rem_ref, out_ref):
        a = lines_ref[...].reshape(_FPAD, 4, _FPAD, 128)
        a = a.transpose(1, 0, 2, 3).reshape(_GROUP, _FPAD, 128)
        r = rem_ref[...].reshape(_GROUP, _FPAD, 1)
        w = jnp.where(
            r < 2,
            jnp.where(r == 0, a[:, :, 0:dim], a[:, :, dim : 2 * dim]),
            jnp.where(r == 2, a[:, :, 2 * dim : 3 * dim], a[:, :, 3 * dim :]),
        )
        out_ref[...] = w[:, :fields, :]

    return pl.pallas_call(
        select_kernel,
        grid=(n_blocks,),
        in_specs=[
            pl.BlockSpec((lines_rows, 128), lambda i: (i, 0)),
            pl.BlockSpec((_GROUP, _FPAD), lambda i: (i, 0)),
        ],
        out_specs=pl.BlockSpec((_GROUP, fields, dim), lambda i: (i, 0, 0)),
        out_shape=jax.ShapeDtypeStruct((batch, fields, dim), jnp.float32),
    )(lines, rem)


def kernel(x, table):
    batch, fields = x.shape
    idx4_2d, rem = _tc_prep(x.astype(jnp.int32))
    table4 = _tc_pack(table.T)
    lines = _sc_gather(table4, idx4_2d)
    return _tc_select(lines, rem, fields)


# speedup vs baseline: 5.4258x; 1.0803x over previous
import functools

import jax
import jax.numpy as jnp
from jax import lax
from jax.experimental import pallas as pl
from jax.experimental.pallas import tpu as pltpu
from jax.experimental.pallas import tpu_sc as plsc

_NUM_CORES = 2
_NUM_SUBCORES = 16
_NUM_WORKERS = _NUM_CORES * _NUM_SUBCORES
_CHUNK = 512
_GROUP = 128
_FPAD = 32


def _tc_prep(x):
    batch, fields = x.shape
    n_blocks = batch // _GROUP

    def prep_kernel(x_ref, idx_ref, rem_ref):
        a = x_ref[...]
        ap = jnp.concatenate([a, a[:, fields - (_FPAD - fields) :]], axis=1)
        rem_ref[...] = (ap >> 10) & 3
        line = ((ap >> 12) << 10) | (ap & 1023)
        idx_ref[...] = jnp.concatenate(
            [line[q : q + _FPAD, :] for q in range(0, _GROUP, _FPAD)], axis=1
        ).reshape(_FPAD, 1, 128)

    return pl.pallas_call(
        prep_kernel,
        grid=(n_blocks,),
        in_specs=[pl.BlockSpec((_GROUP, fields), lambda i: (i, 0))],
        out_specs=[
            pl.BlockSpec((_FPAD, 1, 128), lambda i: (i, 0, 0)),
            pl.BlockSpec((_GROUP, _FPAD), lambda i: (i, 0)),
        ],
        out_shape=[
            jax.ShapeDtypeStruct((batch * _FPAD // 128, 1, 128), jnp.int32),
            jax.ShapeDtypeStruct((batch, _FPAD), jnp.int32),
        ],
    )(x)


_PACK_COLS = 4096


def _tc_pack(tt):
    dim, num_emb = tt.shape
    n_blocks = pl.cdiv(num_emb, _PACK_COLS)
    lines_block = _PACK_COLS // 4

    def pack_kernel(t_ref, out_ref):
        a = t_ref[...].T
        out_ref[...] = jnp.concatenate(
            [a[q : q + lines_block, :] for q in range(0, _PACK_COLS, lines_block)],
            axis=1,
        )

    return pl.pallas_call(
        pack_kernel,
        grid=(n_blocks,),
        in_specs=[pl.BlockSpec((dim, _PACK_COLS), lambda i: (0, i))],
        out_specs=pl.BlockSpec((lines_block, 128), lambda i: (i, 0)),
        out_shape=jax.ShapeDtypeStruct((n_blocks * lines_block, 128), jnp.float32),
    )(tt)


def _sc_gather(table4, idx4_2d):
    idx_rows = idx4_2d.shape[0]
    num_rows = idx_rows * 128
    chunk_idx_rows = _CHUNK // 128
    rows_per_worker = num_rows // _NUM_WORKERS
    n_chunks = rows_per_worker // _CHUNK
    mesh = plsc.VectorSubcoreMesh(core_axis_name="c", subcore_axis_name="s")

    @functools.partial(
        pl.kernel,
        mesh=mesh,
        out_type=jax.ShapeDtypeStruct((num_rows, 128), jnp.float32),
        scratch_types=[
            pltpu.VMEM((8, 1, 128), jnp.int32),
            pltpu.VMEM((_CHUNK, 128), jnp.float32),
            pltpu.SemaphoreType.DMA,
        ],
    )
    def gather_kernel(table_hbm, idx_hbm, out_hbm, idx_v, lines_v, sem):
        wid = lax.axis_index("s") * _NUM_CORES + lax.axis_index("c")
        base = wid * rows_per_worker

        @pl.loop(0, n_chunks // 2)
        def _(c):
            off = base + c * 2 * _CHUNK
            pltpu.sync_copy(idx_hbm.at[pl.ds(off // 128, 8), :, :], idx_v)
            for half in range(2):
                waits = [
                    pltpu.async_copy(
                        table_hbm.at[idx_v.at[4 * half + j, 0]],
                        lines_v.at[pl.ds(j * 128, 128)],
                        sem,
                    )
                    for j in range(chunk_idx_rows)
                ]
                for w in waits:
                    w.wait()
                pltpu.sync_copy(
                    lines_v, out_hbm.at[pl.ds(off + half * _CHUNK, _CHUNK)]
                )

    return gather_kernel(table4, idx4_2d)


def _tc_select(lines, rem, fields):
    batch = rem.shape[0]
    dim = 32
    n_blocks = batch // _GROUP
    lines_rows = _GROUP * _FPAD

    def select_kernel(lines_ref, rem_ref, out_ref):
        a = lines_ref[...].reshape(_FPAD, 4, _FPAD, 128)
        a = a.transpose(1, 0, 2, 3).reshape(_GROUP, _FPAD, 128)
        r = rem_ref[...].reshape(_GROUP, _FPAD, 1)
        w = jnp.where(
            r < 2,
            jnp.where(r == 0, a[:, :, 0:dim], a[:, :, dim : 2 * dim]),
            jnp.where(r == 2, a[:, :, 2 * dim : 3 * dim], a[:, :, 3 * dim :]),
        )
        out_ref[...] = w[:, :fields, :].transpose(1, 2, 0)

    return pl.pallas_call(
        select_kernel,
        grid=(n_blocks,),
        in_specs=[
            pl.BlockSpec((lines_rows, 128), lambda i: (i, 0)),
            pl.BlockSpec((_GROUP, _FPAD), lambda i: (i, 0)),
        ],
        out_specs=pl.BlockSpec((fields, dim, _GROUP), lambda i: (0, 0, i)),
        out_shape=jax.ShapeDtypeStruct((fields, dim, batch), jnp.float32),
    )(lines, rem)


def kernel(x, table):
    batch, fields = x.shape
    idx4_2d, rem = _tc_prep(x.astype(jnp.int32))
    table4 = _tc_pack(table.T)
    lines = _sc_gather(table4, idx4_2d)
    return _tc_select(lines, rem, fields).transpose(2, 0, 1)


# speedup vs baseline: 5.5003x; 1.0137x over previous
import functools

import jax
import jax.numpy as jnp
from jax import lax
from jax.experimental import pallas as pl
from jax.experimental.pallas import tpu as pltpu
from jax.experimental.pallas import tpu_sc as plsc

_NUM_CORES = 2
_NUM_SUBCORES = 16
_NUM_WORKERS = _NUM_CORES * _NUM_SUBCORES
_CHUNK = 256
_GROUP = 128
_FPAD = 32


def _tc_prep(x):
    batch, fields = x.shape
    n_blocks = batch // _GROUP

    def prep_kernel(x_ref, idx_ref, rem_ref):
        a = x_ref[...]
        ap = jnp.concatenate([a, a[:, fields - (_FPAD - fields) :]], axis=1)
        rem_ref[...] = (ap >> 10) & 3
        line = ((ap >> 12) << 10) | (ap & 1023)
        idx_ref[...] = jnp.concatenate(
            [line[q : q + _FPAD, :] for q in range(0, _GROUP, _FPAD)], axis=1
        ).reshape(_FPAD, 1, 128)

    return pl.pallas_call(
        prep_kernel,
        grid=(n_blocks,),
        in_specs=[pl.BlockSpec((_GROUP, fields), lambda i: (i, 0))],
        out_specs=[
            pl.BlockSpec((_FPAD, 1, 128), lambda i: (i, 0, 0)),
            pl.BlockSpec((_GROUP, _FPAD), lambda i: (i, 0)),
        ],
        out_shape=[
            jax.ShapeDtypeStruct((batch * _FPAD // 128, 1, 128), jnp.int32),
            jax.ShapeDtypeStruct((batch, _FPAD), jnp.int32),
        ],
    )(x)


_PACK_COLS = 4096


def _tc_pack(tt):
    dim, num_emb = tt.shape
    n_blocks = pl.cdiv(num_emb, _PACK_COLS)
    lines_block = _PACK_COLS // 4

    def pack_kernel(t_ref, out_ref):
        a = t_ref[...].T
        out_ref[...] = jnp.concatenate(
            [a[q : q + lines_block, :] for q in range(0, _PACK_COLS, lines_block)],
            axis=1,
        )

    return pl.pallas_call(
        pack_kernel,
        grid=(n_blocks,),
        in_specs=[pl.BlockSpec((dim, _PACK_COLS), lambda i: (0, i))],
        out_specs=pl.BlockSpec((lines_block, 128), lambda i: (i, 0)),
        out_shape=jax.ShapeDtypeStruct((n_blocks * lines_block, 128), jnp.float32),
    )(tt)


def _sc_gather(table4, idx4_2d):
    idx_rows = idx4_2d.shape[0]
    num_rows = idx_rows * 128
    chunk_idx_rows = _CHUNK // 128
    rows_per_worker = num_rows // _NUM_WORKERS
    n_chunks = rows_per_worker // _CHUNK
    mesh = plsc.VectorSubcoreMesh(core_axis_name="c", subcore_axis_name="s")

    @functools.partial(
        pl.kernel,
        mesh=mesh,
        out_type=jax.ShapeDtypeStruct((num_rows, 128), jnp.float32),
        scratch_types=[
            pltpu.VMEM((8, 1, 128), jnp.int32),
            pltpu.VMEM((_CHUNK, 128), jnp.float32),
            pltpu.VMEM((_CHUNK, 128), jnp.float32),
            pltpu.SemaphoreType.DMA,
            pltpu.SemaphoreType.DMA,
            pltpu.SemaphoreType.DMA,
        ],
    )
    def gather_kernel(
        table_hbm, idx_hbm, out_hbm, idx_v, lines_a, lines_b, sem_g, sem_wa, sem_wb
    ):
        lines = (lines_a, lines_b)
        sem_w = (sem_wa, sem_wb)
        wid = lax.axis_index("s") * _NUM_CORES + lax.axis_index("c")
        base = wid * rows_per_worker
        n_outer = rows_per_worker // (8 * 128)

        @pl.loop(0, n_outer)
        def _(o):
            obase = base + o * 8 * 128
            pltpu.sync_copy(idx_hbm.at[pl.ds(obase // 128, 8), :, :], idx_v)
            for s in range(4):
                b = s % 2
                off = obase + s * _CHUNK

                def drain(b=b, off=off):
                    pltpu.make_async_copy(
                        lines[b], out_hbm.at[pl.ds(off, _CHUNK)], sem_w[b]
                    ).wait()

                if s >= 2:
                    drain()
                else:
                    pl.when(o > 0)(drain)
                gathers = [
                    pltpu.async_copy(
                        table_hbm.at[idx_v.at[2 * s + j, 0]],
                        lines[b].at[pl.ds(j * 128, 128)],
                        sem_g,
                    )
                    for j in range(2)
                ]
                for g in gathers:
                    g.wait()
                pltpu.async_copy(lines[b], out_hbm.at[pl.ds(off, _CHUNK)], sem_w[b])

        for b in range(2):
            pltpu.make_async_copy(
                lines[b], out_hbm.at[pl.ds(base, _CHUNK)], sem_w[b]
            ).wait()

    return gather_kernel(table4, idx4_2d)


def _tc_select(lines, rem, fields):
    batch = rem.shape[0]
    dim = 32
    n_blocks = batch // _GROUP
    lines_rows = _GROUP * _FPAD

    def select_kernel(lines_ref, rem_ref, out_ref):
        a = lines_ref[...].reshape(_FPAD, 4, _FPAD, 128)
        a = a.transpose(1, 0, 2, 3).reshape(_GROUP, _FPAD, 128)
        r = rem_ref[...].reshape(_GROUP, _FPAD, 1)
        w = jnp.where(
            r < 2,
            jnp.where(r == 0, a[:, :, 0:dim], a[:, :, dim : 2 * dim]),
            jnp.where(r == 2, a[:, :, 2 * dim : 3 * dim], a[:, :, 3 * dim :]),
        )
        out_ref[...] = w[:, :fields, :].transpose(1, 2, 0)

    return pl.pallas_call(
        select_kernel,
        grid=(n_blocks,),
        in_specs=[
            pl.BlockSpec((lines_rows, 128), lambda i: (i, 0)),
            pl.BlockSpec((_GROUP, _FPAD), lambda i: (i, 0)),
        ],
        out_specs=pl.BlockSpec((fields, dim, _GROUP), lambda i: (0, 0, i)),
        out_shape=jax.ShapeDtypeStruct((fields, dim, batch), jnp.float32),
    )(lines, rem)


def kernel(x, table):
    batch, fields = x.shape
    idx4_2d, rem = _tc_prep(x.astype(jnp.int32))
    table4 = _tc_pack(table.T)
    lines = _sc_gather(table4, idx4_2d)
    return _tc_select(lines, rem, fields).transpose(2, 0, 1)


# speedup vs baseline: 5.7668x; 1.0484x over previous
import functools

import jax
import jax.numpy as jnp
from jax import lax
from jax.experimental import pallas as pl
from jax.experimental.pallas import tpu as pltpu
from jax.experimental.pallas import tpu_sc as plsc

_NUM_CORES = 2
_NUM_SUBCORES = 16
_NUM_WORKERS = _NUM_CORES * _NUM_SUBCORES
_CHUNK = 256
_GROUP = 128
_FPAD = 32


def _tc_prep(x):
    batch, fields = x.shape
    n_blocks = batch // _GROUP

    def prep_kernel(x_ref, idx_ref, rem_ref):
        a = x_ref[...]
        ap = jnp.concatenate([a, a[:, fields - (_FPAD - fields) :]], axis=1)
        rem_ref[...] = (ap >> _Q_BITS) & 3
        line = ((ap >> (_Q_BITS + 2)) << _Q_BITS) | (ap & ((1 << _Q_BITS) - 1))
        idx_ref[...] = jnp.concatenate(
            [line[q : q + _FPAD, :] for q in range(0, _GROUP, _FPAD)], axis=1
        ).reshape(_FPAD, 1, 128)

    return pl.pallas_call(
        prep_kernel,
        grid=(n_blocks,),
        in_specs=[pl.BlockSpec((_GROUP, fields), lambda i: (i, 0))],
        out_specs=[
            pl.BlockSpec((_FPAD, 1, 128), lambda i: (i, 0, 0)),
            pl.BlockSpec((_GROUP, _FPAD), lambda i: (i, 0)),
        ],
        out_shape=[
            jax.ShapeDtypeStruct((batch * _FPAD // 128, 1, 128), jnp.int32),
            jax.ShapeDtypeStruct((batch, _FPAD), jnp.int32),
        ],
    )(x)


_PACK_COLS = 8192
_Q_BITS = (_PACK_COLS // 4).bit_length() - 1


def _tc_pack(tt):
    dim, num_emb = tt.shape
    n_blocks = pl.cdiv(num_emb, _PACK_COLS)
    lines_block = _PACK_COLS // 4

    def pack_kernel(t_ref, out_ref):
        a = t_ref[...].T
        out_ref[...] = jnp.concatenate(
            [a[q : q + lines_block, :] for q in range(0, _PACK_COLS, lines_block)],
            axis=1,
        )

    return pl.pallas_call(
        pack_kernel,
        grid=(n_blocks,),
        in_specs=[pl.BlockSpec((dim, _PACK_COLS), lambda i: (0, i))],
        out_specs=pl.BlockSpec((lines_block, 128), lambda i: (i, 0)),
        out_shape=jax.ShapeDtypeStruct((n_blocks * lines_block, 128), jnp.float32),
    )(tt)


def _sc_gather(table4, idx4_2d):
    idx_rows = idx4_2d.shape[0]
    num_rows = idx_rows * 128
    chunk_idx_rows = _CHUNK // 128
    rows_per_worker = num_rows // _NUM_WORKERS
    n_chunks = rows_per_worker // _CHUNK
    mesh = plsc.VectorSubcoreMesh(core_axis_name="c", subcore_axis_name="s")

    @functools.partial(
        pl.kernel,
        mesh=mesh,
        out_type=jax.ShapeDtypeStruct((num_rows, 128), jnp.float32),
        scratch_types=[
            pltpu.VMEM((8, 1, 128), jnp.int32),
            pltpu.VMEM((_CHUNK, 128), jnp.float32),
            pltpu.VMEM((_CHUNK, 128), jnp.float32),
            pltpu.SemaphoreType.DMA,
            pltpu.SemaphoreType.DMA,
            pltpu.SemaphoreType.DMA,
        ],
    )
    def gather_kernel(
        table_hbm, idx_hbm, out_hbm, idx_v, lines_a, lines_b, sem_g, sem_wa, sem_wb
    ):
        lines = (lines_a, lines_b)
        sem_w = (sem_wa, sem_wb)
        wid = lax.axis_index("s") * _NUM_CORES + lax.axis_index("c")
        base = wid * rows_per_worker
        n_outer = rows_per_worker // (8 * 128)

        @pl.loop(0, n_outer)
        def _(o):
            obase = base + o * 8 * 128
            pltpu.sync_copy(idx_hbm.at[pl.ds(obase // 128, 8), :, :], idx_v)
            for s in range(4):
                b = s % 2
                off = obase + s * _CHUNK

                def drain(b=b, off=off):
                    pltpu.make_async_copy(
                        lines[b], out_hbm.at[pl.ds(off, _CHUNK)], sem_w[b]
                    ).wait()

                if s >= 2:
                    drain()
                else:
                    pl.when(o > 0)(drain)
                gathers = [
                    pltpu.async_copy(
                        table_hbm.at[idx_v.at[2 * s + j, 0]],
                        lines[b].at[pl.ds(j * 128, 128)],
                        sem_g,
                    )
                    for j in range(2)
                ]
                for g in gathers:
                    g.wait()
                pltpu.async_copy(lines[b], out_hbm.at[pl.ds(off, _CHUNK)], sem_w[b])

        for b in range(2):
            pltpu.make_async_copy(
                lines[b], out_hbm.at[pl.ds(base, _CHUNK)], sem_w[b]
            ).wait()

    return gather_kernel(table4, idx4_2d)


def _tc_select(lines, rem, fields):
    batch = rem.shape[0]
    dim = 32
    n_blocks = batch // _GROUP
    lines_rows = _GROUP * _FPAD

    def select_kernel(lines_ref, rem_ref, out_ref):
        a = lines_ref[...].reshape(_FPAD, 4, _FPAD, 128)
        a = a.transpose(1, 0, 2, 3).reshape(_GROUP, _FPAD, 128)
        r = rem_ref[...].reshape(_GROUP, _FPAD, 1)
        w = jnp.where(
            r < 2,
            jnp.where(r == 0, a[:, :, 0:dim], a[:, :, dim : 2 * dim]),
            jnp.where(r == 2, a[:, :, 2 * dim : 3 * dim], a[:, :, 3 * dim :]),
        )
        out_ref[...] = w.transpose(1, 2, 0)[:fields]

    return pl.pallas_call(
        select_kernel,
        grid=(n_blocks,),
        in_specs=[
            pl.BlockSpec((lines_rows, 128), lambda i: (i, 0)),
            pl.BlockSpec((_GROUP, _FPAD), lambda i: (i, 0)),
        ],
        out_specs=pl.BlockSpec((fields, dim, _GROUP), lambda i: (0, 0, i)),
        out_shape=jax.ShapeDtypeStruct((fields, dim, batch), jnp.float32),
    )(lines, rem)


def kernel(x, table):
    batch, fields = x.shape
    idx4_2d, rem = _tc_prep(x.astype(jnp.int32))
    table4 = _tc_pack(table.T)
    lines = _sc_gather(table4, idx4_2d)
    return _tc_select(lines, rem, fields).transpose(2, 0, 1)


# speedup vs baseline: 6.0328x; 1.0461x over previous
import functools

import jax
import jax.numpy as jnp
from jax import lax
from jax.experimental import pallas as pl
from jax.experimental.pallas import tpu as pltpu
from jax.experimental.pallas import tpu_sc as plsc

_NUM_CORES = 2
_NUM_SUBCORES = 16
_NUM_WORKERS = _NUM_CORES * _NUM_SUBCORES
_CHUNK = 256
_GROUP = 128
_FPAD = 32


def _tc_prep(x):
    batch, fields = x.shape
    n_blocks = batch // _GROUP

    def prep_kernel(x_ref, idx_ref, rem_ref):
        a = x_ref[...]
        ap = jnp.concatenate([a, a[:, fields - (_FPAD - fields) :]], axis=1)
        rem_ref[...] = (ap >> _Q_BITS) & 3
        line = ((ap >> (_Q_BITS + 2)) << _Q_BITS) | (ap & ((1 << _Q_BITS) - 1))
        idx_ref[...] = jnp.concatenate(
            [line[q : q + _FPAD, :] for q in range(0, _GROUP, _FPAD)], axis=1
        ).reshape(_FPAD, 1, 128)

    return pl.pallas_call(
        prep_kernel,
        grid=(n_blocks,),
        in_specs=[pl.BlockSpec((_GROUP, fields), lambda i: (i, 0))],
        out_specs=[
            pl.BlockSpec((_FPAD, 1, 128), lambda i: (i, 0, 0)),
            pl.BlockSpec((_GROUP, _FPAD), lambda i: (i, 0)),
        ],
        out_shape=[
            jax.ShapeDtypeStruct((batch * _FPAD // 128, 1, 128), jnp.int32),
            jax.ShapeDtypeStruct((batch, _FPAD), jnp.int32),
        ],
    )(x)


_PACK_COLS = 8192
_Q_BITS = (_PACK_COLS // 4).bit_length() - 1


def _tc_pack(tt):
    dim, num_emb = tt.shape
    n_blocks = pl.cdiv(num_emb, _PACK_COLS)
    lines_block = _PACK_COLS // 4

    def pack_kernel(t_ref, out_ref):
        a = t_ref[...].T
        out_ref[...] = jnp.concatenate(
            [a[q : q + lines_block, :] for q in range(0, _PACK_COLS, lines_block)],
            axis=1,
        )

    return pl.pallas_call(
        pack_kernel,
        grid=(n_blocks,),
        in_specs=[pl.BlockSpec((dim, _PACK_COLS), lambda i: (0, i))],
        out_specs=pl.BlockSpec((lines_block, 128), lambda i: (i, 0)),
        out_shape=jax.ShapeDtypeStruct((n_blocks * lines_block, 128), jnp.float32),
    )(tt)


def _sc_gather(table4, idx4_2d):
    idx_rows = idx4_2d.shape[0]
    num_rows = idx_rows * 128
    chunk_idx_rows = _CHUNK // 128
    rows_per_worker = num_rows // _NUM_WORKERS
    n_chunks = rows_per_worker // _CHUNK
    mesh = plsc.VectorSubcoreMesh(core_axis_name="c", subcore_axis_name="s")

    @functools.partial(
        pl.kernel,
        mesh=mesh,
        out_type=jax.ShapeDtypeStruct((num_rows, 128), jnp.float32),
        scratch_types=[
            pltpu.VMEM((8, 1, 128), jnp.int32),
            pltpu.VMEM((_CHUNK, 128), jnp.float32),
            pltpu.VMEM((_CHUNK, 128), jnp.float32),
            pltpu.SemaphoreType.DMA,
            pltpu.SemaphoreType.DMA,
            pltpu.SemaphoreType.DMA,
        ],
    )
    def gather_kernel(
        table_hbm, idx_hbm, out_hbm, idx_v, lines_a, lines_b, sem_g, sem_wa, sem_wb
    ):
        lines = (lines_a, lines_b)
        sem_w = (sem_wa, sem_wb)
        wid = lax.axis_index("s") * _NUM_CORES + lax.axis_index("c")
        base = wid * rows_per_worker
        n_outer = rows_per_worker // (8 * 128)

        @pl.loop(0, n_outer)
        def _(o):
            obase = base + o * 8 * 128
            pltpu.sync_copy(idx_hbm.at[pl.ds(obase // 128, 8), :, :], idx_v)
            for s in range(4):
                b = s % 2
                off = obase + s * _CHUNK

                def drain(b=b, off=off):
                    pltpu.make_async_copy(
                        lines[b], out_hbm.at[pl.ds(off, _CHUNK)], sem_w[b]
                    ).wait()

                if s >= 2:
                    drain()
                else:
                    pl.when(o > 0)(drain)
                gathers = [
                    pltpu.async_copy(
                        table_hbm.at[idx_v.at[2 * s + j, 0]],
                        lines[b].at[pl.ds(j * 128, 128)],
                        sem_g,
                    )
                    for j in range(2)
                ]
                for g in gathers:
                    g.wait()
                pltpu.async_copy(lines[b], out_hbm.at[pl.ds(off, _CHUNK)], sem_w[b])

        for b in range(2):
            pltpu.make_async_copy(
                lines[b], out_hbm.at[pl.ds(base, _CHUNK)], sem_w[b]
            ).wait()

    return gather_kernel(table4, idx4_2d)


def _tc_select(lines, rem, fields):
    batch = rem.shape[0]
    dim = 32
    n_blocks = batch // _GROUP
    lines_rows = _GROUP * _FPAD

    def select_kernel(lines_ref, rem_ref, out_ref):
        a = lines_ref[...].reshape(_FPAD, 4, _FPAD, 128)
        a = a.transpose(1, 0, 2, 3).reshape(_GROUP, _FPAD, 128)
        r = rem_ref[...].reshape(_GROUP, _FPAD, 1)
        w = jnp.where(
            r < 2,
            jnp.where(r == 0, a[:, :, 0:dim], a[:, :, dim : 2 * dim]),
            jnp.where(r == 2, a[:, :, 2 * dim : 3 * dim], a[:, :, 3 * dim :]),
        )
        out_ref[...] = w.transpose(1, 2, 0)[:fields]

    return pl.pallas_call(
        select_kernel,
        grid=(n_blocks,),
        in_specs=[
            pl.BlockSpec((lines_rows, 128), lambda i: (i, 0)),
            pl.BlockSpec((_GROUP, _FPAD), lambda i: (i, 0)),
        ],
        out_specs=pl.BlockSpec((fields, dim, _GROUP), lambda i: (0, 0, i)),
        out_shape=jax.ShapeDtypeStruct((fields, dim, batch), jnp.float32),
    )(lines, rem)


def kernel(x, table):
    batch, fields = x.shape
    idx4_2d, rem = _tc_prep(x.astype(jnp.int32))
    table4 = _tc_pack(table.T)
    n_slices = 2
    ib = idx4_2d.shape[0] // n_slices
    rb = batch // n_slices
    outs = []
    for k in range(n_slices):
        lines = _sc_gather(table4, idx4_2d[k * ib : (k + 1) * ib])
        outs.append(
            _tc_select(lines, rem[k * rb : (k + 1) * rb], fields)
        )
    return jnp.concatenate(outs, axis=2).transpose(2, 0, 1)


# speedup vs baseline: 6.2547x; 1.0368x over previous
import functools

import jax
import jax.numpy as jnp
from jax import lax
from jax.experimental import pallas as pl
from jax.experimental.pallas import tpu as pltpu
from jax.experimental.pallas import tpu_sc as plsc

_NUM_CORES = 2
_NUM_SUBCORES = 16
_NUM_WORKERS = _NUM_CORES * _NUM_SUBCORES
_CHUNK = 256
_GROUP = 128
_FPAD = 32


def _tc_prep(x):
    batch, fields = x.shape
    n_blocks = batch // _GROUP

    def prep_kernel(x_ref, idx_ref, rem_ref):
        a = x_ref[...]
        ap = jnp.concatenate([a, a[:, fields - (_FPAD - fields) :]], axis=1)
        rem_ref[...] = (ap >> _Q_BITS) & 3
        line = ((ap >> (_Q_BITS + 2)) << _Q_BITS) | (ap & ((1 << _Q_BITS) - 1))
        idx_ref[...] = jnp.concatenate(
            [line[q : q + _FPAD, :] for q in range(0, _GROUP, _FPAD)], axis=1
        ).reshape(_FPAD, 1, 128)

    return pl.pallas_call(
        prep_kernel,
        grid=(n_blocks,),
        in_specs=[pl.BlockSpec((_GROUP, fields), lambda i: (i, 0))],
        out_specs=[
            pl.BlockSpec((_FPAD, 1, 128), lambda i: (i, 0, 0)),
            pl.BlockSpec((_GROUP, _FPAD), lambda i: (i, 0)),
        ],
        out_shape=[
            jax.ShapeDtypeStruct((batch * _FPAD // 128, 1, 128), jnp.int32),
            jax.ShapeDtypeStruct((batch, _FPAD), jnp.int32),
        ],
    )(x)


_PACK_COLS = 8192
_Q_BITS = (_PACK_COLS // 4).bit_length() - 1


def _tc_pack(tt):
    dim, num_emb = tt.shape
    n_blocks = pl.cdiv(num_emb, _PACK_COLS)
    lines_block = _PACK_COLS // 4

    def pack_kernel(t_ref, out_ref):
        a = t_ref[...].T
        out_ref[...] = jnp.concatenate(
            [a[q : q + lines_block, :] for q in range(0, _PACK_COLS, lines_block)],
            axis=1,
        )

    return pl.pallas_call(
        pack_kernel,
        grid=(n_blocks,),
        in_specs=[pl.BlockSpec((dim, _PACK_COLS), lambda i: (0, i))],
        out_specs=pl.BlockSpec((lines_block, 128), lambda i: (i, 0)),
        out_shape=jax.ShapeDtypeStruct((n_blocks * lines_block, 128), jnp.float32),
    )(tt)


def _sc_gather(table4, idx4_2d):
    idx_rows = idx4_2d.shape[0]
    num_rows = idx_rows * 128
    chunk_idx_rows = _CHUNK // 128
    rows_per_worker = num_rows // _NUM_WORKERS
    n_chunks = rows_per_worker // _CHUNK
    mesh = plsc.VectorSubcoreMesh(core_axis_name="c", subcore_axis_name="s")

    @functools.partial(
        pl.kernel,
        mesh=mesh,
        out_type=jax.ShapeDtypeStruct((num_rows, 128), jnp.float32),
        scratch_types=[
            pltpu.VMEM((8, 1, 128), jnp.int32),
            pltpu.VMEM((_CHUNK, 128), jnp.float32),
            pltpu.VMEM((_CHUNK, 128), jnp.float32),
            pltpu.SemaphoreType.DMA,
            pltpu.SemaphoreType.DMA,
            pltpu.SemaphoreType.DMA,
        ],
    )
    def gather_kernel(
        table_hbm, idx_hbm, out_hbm, idx_v, lines_a, lines_b, sem_g, sem_wa, sem_wb
    ):
        lines = (lines_a, lines_b)
        sem_w = (sem_wa, sem_wb)
        wid = lax.axis_index("s") * _NUM_CORES + lax.axis_index("c")
        base = wid * rows_per_worker
        n_outer = rows_per_worker // (8 * 128)

        @pl.loop(0, n_outer)
        def _(o):
            obase = base + o * 8 * 128
            pltpu.sync_copy(idx_hbm.at[pl.ds(obase // 128, 8), :, :], idx_v)
            for s in range(4):
                b = s % 2
                off = obase + s * _CHUNK

                def drain(b=b, off=off):
                    pltpu.make_async_copy(
                        lines[b], out_hbm.at[pl.ds(off, _CHUNK)], sem_w[b]
                    ).wait()

                if s >= 2:
                    drain()
                else:
                    pl.when(o > 0)(drain)
                gathers = [
                    pltpu.async_copy(
                        table_hbm.at[idx_v.at[2 * s + j, 0]],
                        lines[b].at[pl.ds(j * 128, 128)],
                        sem_g,
                    )
                    for j in range(2)
                ]
                for g in gathers:
                    g.wait()
                pltpu.async_copy(lines[b], out_hbm.at[pl.ds(off, _CHUNK)], sem_w[b])

        for b in range(2):
            pltpu.make_async_copy(
                lines[b], out_hbm.at[pl.ds(base, _CHUNK)], sem_w[b]
            ).wait()

    return gather_kernel(table4, idx4_2d)


def _tc_select(lines, rem, fields):
    batch = rem.shape[0]
    dim = 32
    n_blocks = batch // _GROUP
    lines_rows = _GROUP * _FPAD

    def select_kernel(lines_ref, rem_ref, out_ref):
        a = lines_ref[...].reshape(_FPAD, 4, _FPAD, 128)
        a = a.transpose(1, 0, 2, 3).reshape(_GROUP, _FPAD, 128)
        r = rem_ref[...].reshape(_GROUP, _FPAD, 1)
        w = jnp.where(
            r < 2,
            jnp.where(r == 0, a[:, :, 0:dim], a[:, :, dim : 2 * dim]),
            jnp.where(r == 2, a[:, :, 2 * dim : 3 * dim], a[:, :, 3 * dim :]),
        )
        out_ref[...] = w.transpose(1, 2, 0)[:fields]

    return pl.pallas_call(
        select_kernel,
        grid=(n_blocks,),
        in_specs=[
            pl.BlockSpec((lines_rows, 128), lambda i: (i, 0)),
            pl.BlockSpec((_GROUP, _FPAD), lambda i: (i, 0)),
        ],
        out_specs=pl.BlockSpec((fields, dim, _GROUP), lambda i: (0, 0, i)),
        out_shape=jax.ShapeDtypeStruct((fields, dim, batch), jnp.float32),
    )(lines, rem)


def kernel(x, table):
    batch, fields = x.shape
    idx4_2d, rem = _tc_prep(x.astype(jnp.int32))
    table4 = _tc_pack(table.T)
    n_slices = 4
    ib = idx4_2d.shape[0] // n_slices
    rb = batch // n_slices
    outs = []
    for k in range(n_slices):
        lines = _sc_gather(table4, idx4_2d[k * ib : (k + 1) * ib])
        outs.append(
            _tc_select(lines, rem[k * rb : (k + 1) * rb], fields)
        )
    return jnp.concatenate(outs, axis=2).transpose(2, 0, 1)


# speedup vs baseline: 6.2671x; 1.0020x over previous
import functools

import jax
import jax.numpy as jnp
from jax import lax
from jax.experimental import pallas as pl
from jax.experimental.pallas import tpu as pltpu
from jax.experimental.pallas import tpu_sc as plsc

_NUM_CORES = 2
_NUM_SUBCORES = 16
_NUM_WORKERS = _NUM_CORES * _NUM_SUBCORES
_CHUNK = 256
_GROUP = 128
_FPAD = 32


def _tc_prep(x):
    batch, fields = x.shape
    n_blocks = batch // _GROUP

    def prep_kernel(x_ref, idx_ref, rem_ref):
        a = x_ref[...]
        ap = jnp.concatenate([a, a[:, fields - (_FPAD - fields) :]], axis=1)
        rem_ref[...] = (ap >> _Q_BITS) & 3
        line = ((ap >> (_Q_BITS + 2)) << _Q_BITS) | (ap & ((1 << _Q_BITS) - 1))
        idx_ref[...] = jnp.concatenate(
            [line[q : q + _FPAD, :] for q in range(0, _GROUP, _FPAD)], axis=1
        ).reshape(_FPAD, 1, 128)

    return pl.pallas_call(
        prep_kernel,
        grid=(n_blocks,),
        in_specs=[pl.BlockSpec((_GROUP, fields), lambda i: (i, 0))],
        out_specs=[
            pl.BlockSpec((_FPAD, 1, 128), lambda i: (i, 0, 0)),
            pl.BlockSpec((_GROUP, _FPAD), lambda i: (i, 0)),
        ],
        out_shape=[
            jax.ShapeDtypeStruct((batch * _FPAD // 128, 1, 128), jnp.int32),
            jax.ShapeDtypeStruct((batch, _FPAD), jnp.int32),
        ],
    )(x)


_PACK_COLS = 8192
_Q_BITS = (_PACK_COLS // 4).bit_length() - 1


def _tc_pack(tt):
    dim, num_emb = tt.shape
    n_blocks = pl.cdiv(num_emb, _PACK_COLS)
    lines_block = _PACK_COLS // 4

    def pack_kernel(t_ref, out_ref):
        a = t_ref[...].T
        out_ref[...] = jnp.concatenate(
            [a[q : q + lines_block, :] for q in range(0, _PACK_COLS, lines_block)],
            axis=1,
        )

    return pl.pallas_call(
        pack_kernel,
        grid=(n_blocks,),
        in_specs=[pl.BlockSpec((dim, _PACK_COLS), lambda i: (0, i))],
        out_specs=pl.BlockSpec((lines_block, 128), lambda i: (i, 0)),
        out_shape=jax.ShapeDtypeStruct((n_blocks * lines_block, 128), jnp.float32),
    )(tt)


def _sc_gather(table4, idx4_2d):
    idx_rows = idx4_2d.shape[0]
    num_rows = idx_rows * 128
    chunk_idx_rows = _CHUNK // 128
    rows_per_worker = num_rows // _NUM_WORKERS
    n_chunks = rows_per_worker // _CHUNK
    mesh = plsc.VectorSubcoreMesh(core_axis_name="c", subcore_axis_name="s")

    @functools.partial(
        pl.kernel,
        mesh=mesh,
        out_type=jax.ShapeDtypeStruct((num_rows, 128), jnp.float32),
        scratch_types=[
            pltpu.VMEM((8, 1, 128), jnp.int32),
            pltpu.VMEM((_CHUNK, 128), jnp.float32),
            pltpu.VMEM((_CHUNK, 128), jnp.float32),
            pltpu.SemaphoreType.DMA,
            pltpu.SemaphoreType.DMA,
            pltpu.SemaphoreType.DMA,
        ],
    )
    def gather_kernel(
        table_hbm, idx_hbm, out_hbm, idx_v, lines_a, lines_b, sem_g, sem_wa, sem_wb
    ):
        lines = (lines_a, lines_b)
        sem_w = (sem_wa, sem_wb)
        wid = lax.axis_index("s") * _NUM_CORES + lax.axis_index("c")
        base = wid * rows_per_worker
        n_outer = rows_per_worker // (8 * 128)

        @pl.loop(0, n_outer)
        def _(o):
            obase = base + o * 8 * 128
            pltpu.sync_copy(idx_hbm.at[pl.ds(obase // 128, 8), :, :], idx_v)
            for s in range(4):
                b = s % 2
                off = obase + s * _CHUNK

                def drain(b=b, off=off):
                    pltpu.make_async_copy(
                        lines[b], out_hbm.at[pl.ds(off, _CHUNK)], sem_w[b]
                    ).wait()

                if s >= 2:
                    drain()
                else:
                    pl.when(o > 0)(drain)
                gathers = [
                    pltpu.async_copy(
                        table_hbm.at[idx_v.at[2 * s + j, 0]],
                        lines[b].at[pl.ds(j * 128, 128)],
                        sem_g,
                    )
                    for j in range(2)
                ]
                for g in gathers:
                    g.wait()
                pltpu.async_copy(lines[b], out_hbm.at[pl.ds(off, _CHUNK)], sem_w[b])

        for b in range(2):
            pltpu.make_async_copy(
                lines[b], out_hbm.at[pl.ds(base, _CHUNK)], sem_w[b]
            ).wait()

    return gather_kernel(table4, idx4_2d)


def _tc_select(lines, rem, fields):
    batch = rem.shape[0]
    dim = 32
    n_blocks = batch // _GROUP
    lines_rows = _GROUP * _FPAD

    def select_kernel(lines_ref, rem_ref, out_ref):
        a = lines_ref[...].reshape(_FPAD, 4, _FPAD, 128)
        a = a.transpose(1, 0, 2, 3).reshape(_GROUP, _FPAD, 128)
        r = rem_ref[...].reshape(_GROUP, _FPAD, 1)
        w = jnp.where(
            r < 2,
            jnp.where(r == 0, a[:, :, 0:dim], a[:, :, dim : 2 * dim]),
            jnp.where(r == 2, a[:, :, 2 * dim : 3 * dim], a[:, :, 3 * dim :]),
        )
        out_ref[...] = w.transpose(1, 2, 0)[:fields]

    return pl.pallas_call(
        select_kernel,
        grid=(n_blocks,),
        in_specs=[
            pl.BlockSpec((lines_rows, 128), lambda i: (i, 0)),
            pl.BlockSpec((_GROUP, _FPAD), lambda i: (i, 0)),
        ],
        out_specs=pl.BlockSpec((fields, dim, _GROUP), lambda i: (0, 0, i)),
        out_shape=jax.ShapeDtypeStruct((fields, dim, batch), jnp.float32),
    )(lines, rem)


def kernel(x, table):
    batch, fields = x.shape
    idx4_2d, rem = _tc_prep(x.astype(jnp.int32))
    table4 = _tc_pack(table.T)
    n_slices = 8
    ib = idx4_2d.shape[0] // n_slices
    rb = batch // n_slices
    outs = []
    for k in range(n_slices):
        lines = _sc_gather(table4, idx4_2d[k * ib : (k + 1) * ib])
        outs.append(
            _tc_select(lines, rem[k * rb : (k + 1) * rb], fields)
        )
    return jnp.concatenate(outs, axis=2).transpose(2, 0, 1)


# speedup vs baseline: 6.2823x; 1.0024x over previous
import functools

import jax
import jax.numpy as jnp
from jax import lax
from jax.experimental import pallas as pl
from jax.experimental.pallas import tpu as pltpu
from jax.experimental.pallas import tpu_sc as plsc

_NUM_CORES = 2
_NUM_SUBCORES = 16
_NUM_WORKERS = _NUM_CORES * _NUM_SUBCORES
_CHUNK = 256
_GROUP = 128
_FPAD = 32


def _tc_prep(x):
    batch, fields = x.shape
    n_blocks = batch // _GROUP

    def prep_kernel(x_ref, idx_ref, rem_ref):
        a = x_ref[...]
        ap = jnp.concatenate([a, a[:, fields - (_FPAD - fields) :]], axis=1)
        rem_ref[...] = (ap >> _Q_BITS) & 3
        line = ((ap >> (_Q_BITS + 2)) << _Q_BITS) | (ap & ((1 << _Q_BITS) - 1))
        idx_ref[...] = jnp.concatenate(
            [line[q : q + _FPAD, :] for q in range(0, _GROUP, _FPAD)], axis=1
        ).reshape(_FPAD, 1, 128)

    return pl.pallas_call(
        prep_kernel,
        grid=(n_blocks,),
        in_specs=[pl.BlockSpec((_GROUP, fields), lambda i: (i, 0))],
        out_specs=[
            pl.BlockSpec((_FPAD, 1, 128), lambda i: (i, 0, 0)),
            pl.BlockSpec((_GROUP, _FPAD), lambda i: (i, 0)),
        ],
        out_shape=[
            jax.ShapeDtypeStruct((batch * _FPAD // 128, 1, 128), jnp.int32),
            jax.ShapeDtypeStruct((batch, _FPAD), jnp.int32),
        ],
    )(x)


_PACK_COLS = 16384
_Q_BITS = (_PACK_COLS // 4).bit_length() - 1


def _tc_pack(tt):
    dim, num_emb = tt.shape
    n_blocks = pl.cdiv(num_emb, _PACK_COLS)
    lines_block = _PACK_COLS // 4

    def pack_kernel(t_ref, out_ref):
        a = t_ref[...].T
        out_ref[...] = jnp.concatenate(
            [a[q : q + lines_block, :] for q in range(0, _PACK_COLS, lines_block)],
            axis=1,
        )

    return pl.pallas_call(
        pack_kernel,
        grid=(n_blocks,),
        in_specs=[pl.BlockSpec((dim, _PACK_COLS), lambda i: (0, i))],
        out_specs=pl.BlockSpec((lines_block, 128), lambda i: (i, 0)),
        out_shape=jax.ShapeDtypeStruct((n_blocks * lines_block, 128), jnp.float32),
    )(tt)


def _sc_gather(table4, idx4_2d):
    idx_rows = idx4_2d.shape[0]
    num_rows = idx_rows * 128
    chunk_idx_rows = _CHUNK // 128
    rows_per_worker = num_rows // _NUM_WORKERS
    n_chunks = rows_per_worker // _CHUNK
    mesh = plsc.VectorSubcoreMesh(core_axis_name="c", subcore_axis_name="s")

    @functools.partial(
        pl.kernel,
        mesh=mesh,
        out_type=jax.ShapeDtypeStruct((num_rows, 128), jnp.float32),
        scratch_types=[
            pltpu.VMEM((8, 1, 128), jnp.int32),
            pltpu.VMEM((_CHUNK, 128), jnp.float32),
            pltpu.VMEM((_CHUNK, 128), jnp.float32),
            pltpu.SemaphoreType.DMA,
            pltpu.SemaphoreType.DMA,
            pltpu.SemaphoreType.DMA,
        ],
    )
    def gather_kernel(
        table_hbm, idx_hbm, out_hbm, idx_v, lines_a, lines_b, sem_g, sem_wa, sem_wb
    ):
        lines = (lines_a, lines_b)
        sem_w = (sem_wa, sem_wb)
        wid = lax.axis_index("s") * _NUM_CORES + lax.axis_index("c")
        base = wid * rows_per_worker
        n_outer = rows_per_worker // (8 * 128)

        @pl.loop(0, n_outer)
        def _(o):
            obase = base + o * 8 * 128
            pltpu.sync_copy(idx_hbm.at[pl.ds(obase // 128, 8), :, :], idx_v)
            for s in range(4):
                b = s % 2
                off = obase + s * _CHUNK

                def drain(b=b, off=off):
                    pltpu.make_async_copy(
                        lines[b], out_hbm.at[pl.ds(off, _CHUNK)], sem_w[b]
                    ).wait()

                if s >= 2:
                    drain()
                else:
                    pl.when(o > 0)(drain)
                gathers = [
                    pltpu.async_copy(
                        table_hbm.at[idx_v.at[2 * s + j, 0]],
                        lines[b].at[pl.ds(j * 128, 128)],
                        sem_g,
                    )
                    for j in range(2)
                ]
                for g in gathers:
                    g.wait()
                pltpu.async_copy(lines[b], out_hbm.at[pl.ds(off, _CHUNK)], sem_w[b])

        for b in range(2):
            pltpu.make_async_copy(
                lines[b], out_hbm.at[pl.ds(base, _CHUNK)], sem_w[b]
            ).wait()

    return gather_kernel(table4, idx4_2d)


def _tc_select(lines, rem, fields):
    batch = rem.shape[0]
    dim = 32
    n_blocks = batch // _GROUP
    lines_rows = _GROUP * _FPAD

    def select_kernel(lines_ref, rem_ref, out_ref):
        a = lines_ref[...].reshape(_FPAD, 4, _FPAD, 128)
        a = a.transpose(1, 0, 2, 3).reshape(_GROUP, _FPAD, 128)
        r = rem_ref[...].reshape(_GROUP, _FPAD, 1)
        w = jnp.where(
            r < 2,
            jnp.where(r == 0, a[:, :, 0:dim], a[:, :, dim : 2 * dim]),
            jnp.where(r == 2, a[:, :, 2 * dim : 3 * dim], a[:, :, 3 * dim :]),
        )
        out_ref[...] = w.transpose(1, 2, 0)[:fields]

    return pl.pallas_call(
        select_kernel,
        grid=(n_blocks,),
        in_specs=[
            pl.BlockSpec((lines_rows, 128), lambda i: (i, 0)),
            pl.BlockSpec((_GROUP, _FPAD), lambda i: (i, 0)),
        ],
        out_specs=pl.BlockSpec((fields, dim, _GROUP), lambda i: (0, 0, i)),
        out_shape=jax.ShapeDtypeStruct((fields, dim, batch), jnp.float32),
    )(lines, rem)


def kernel(x, table):
    batch, fields = x.shape
    idx4_2d, rem = _tc_prep(x.astype(jnp.int32))
    table4 = _tc_pack(table.T)
    n_slices = 8
    ib = idx4_2d.shape[0] // n_slices
    rb = batch // n_slices
    outs = []
    for k in range(n_slices):
        lines = _sc_gather(table4, idx4_2d[k * ib : (k + 1) * ib])
        outs.append(
            _tc_select(lines, rem[k * rb : (k + 1) * rb], fields)
        )
    return jnp.concatenate(outs, axis=2).transpose(2, 0, 1)
